# baseline (speedup 1.0000x reference)
import functools

import jax
import jax.numpy as jnp
from jax.experimental import pallas as pl
from jax.experimental.pallas import tpu as pltpu


def _direct_body(xf_ref, w_ref, sb_ref, o_ref, *, taps_a, groups, gshift,
                 co4, row_stride, out_len, relu):
    bb = xf_ref.shape[0]
    for b in range(bb):
        pc = jnp.dot(w_ref[...], xf_ref[b],
                     preferred_element_type=jnp.float32)
        acc = None
        for a in range(taps_a):
            for g in range(groups):
                r0 = (a * groups + g) * co4
                off = a * row_stride + g * gshift
                term = pc[r0:r0 + co4, off: off + out_len]
                acc = term if acc is None else acc + term
        y = acc * sb_ref[:, 0:1] + sb_ref[:, 1:2]
        if relu:
            y = jnp.maximum(y, 0.0)
        o_ref[b] = y.astype(o_ref.dtype)


def _direct_call(xf, wm, sb, *, taps_a, groups, gshift, row_stride, out_len,
                 relu, out_dtype, bb):
    B, K, L = xf.shape
    M = wm.shape[0]
    co4 = M // (taps_a * groups)
    body = functools.partial(_direct_body, taps_a=taps_a, groups=groups,
                             gshift=gshift, co4=co4, row_stride=row_stride,
                             out_len=out_len, relu=relu)
    return pl.pallas_call(
        body,
        out_shape=jax.ShapeDtypeStruct((B, co4, out_len), out_dtype),
        grid=(B // bb,),
        in_specs=[
            pl.BlockSpec((bb, K, L), lambda i: (i, 0, 0)),
            pl.BlockSpec((M, K), lambda i: (0, 0)),
            pl.BlockSpec((co4, 2), lambda i: (0, 0)),
        ],
        out_specs=pl.BlockSpec((bb, co4, out_len), lambda i: (i, 0, 0)),
        compiler_params=pltpu.CompilerParams(
            dimension_semantics=("parallel",)),
    )(xf, wm, sb)


def _scratch_body(xs_ref, w_ref, sb_ref, o_ref, xr_ref, *, taps, cdepth,
                  co4, oh, ws, nb, relu):
    ll = oh * ws
    lt = ll + taps - 1
    bb = xs_ref.shape[0]
    for b in range(bb):
        for t in range(taps):
            xr_ref[t * cdepth:(t + 1) * cdepth, :] = (
                xs_ref[b, :, t * ws: t * ws + lt])
        n0 = 0
        while n0 < ll:
            nw = min(nb, ll - n0)
            pc = jnp.dot(w_ref[...], xr_ref[:, n0: n0 + nw + taps - 1],
                         preferred_element_type=jnp.float32)
            acc = pc[0:co4, 0:nw]
            for c in range(1, taps):
                acc = acc + pc[c * co4:(c + 1) * co4, c: c + nw]
            y = acc * sb_ref[:, 0:1] + sb_ref[:, 1:2]
            if relu:
                y = jnp.maximum(y, 0.0)
            o_ref[b, :, n0: n0 + nw] = y.astype(o_ref.dtype)
            n0 += nw


def _scratch_call(xs, wm, sb, *, taps, oh, ws, relu, out_dtype, bb):
    B, C, FL = xs.shape
    M, K = wm.shape
    co4 = M // taps
    ll = oh * ws
    lt = ll + taps - 1
    nb = max(128, (600_000 // (M * 4)) // 128 * 128)
    body = functools.partial(_scratch_body, taps=taps, cdepth=C, co4=co4,
                             oh=oh, ws=ws, nb=nb, relu=relu)
    return pl.pallas_call(
        body,
        out_shape=jax.ShapeDtypeStruct((B, co4, ll), out_dtype),
        grid=(B // bb,),
        in_specs=[
            pl.BlockSpec((bb, C, FL), lambda i: (i, 0, 0)),
            pl.BlockSpec((M, K), lambda i: (0, 0)),
            pl.BlockSpec((co4, 2), lambda i: (0, 0)),
        ],
        out_specs=pl.BlockSpec((bb, co4, ll), lambda i: (i, 0, 0)),
        scratch_shapes=[pltpu.VMEM((K, lt), jnp.bfloat16)],
        compiler_params=pltpu.CompilerParams(
            dimension_semantics=("parallel",)),
    )(xs, wm, sb)


def _mlp_body(z_ref, w1_ref, sb1_ref, w2_ref, b2_ref, o_ref):
    h = jnp.dot(z_ref[...], w1_ref[...], preferred_element_type=jnp.float32)
    h = h * sb1_ref[0:1, :] + sb1_ref[1:2, :]
    h = jnp.maximum(h, 0.0).astype(w2_ref.dtype)
    o_ref[...] = jnp.dot(h, w2_ref[...],
                         preferred_element_type=jnp.float32) + b2_ref[...]


def _mlp_call(z, w1t, sb1, w2t, b2):
    B = z.shape[0]
    gb = B // 2 if B % 2 == 0 else B
    return pl.pallas_call(
        _mlp_body,
        out_shape=jax.ShapeDtypeStruct((B, w2t.shape[1]), jnp.float32),
        grid=(B // gb,),
        in_specs=[
            pl.BlockSpec((gb, w1t.shape[0]), lambda i: (i, 0)),
            pl.BlockSpec(w1t.shape, lambda i: (0, 0)),
            pl.BlockSpec(sb1.shape, lambda i: (0, 0)),
            pl.BlockSpec(w2t.shape, lambda i: (0, 0)),
            pl.BlockSpec(b2.shape, lambda i: (0, 0)),
        ],
        out_specs=pl.BlockSpec((gb, w2t.shape[1]), lambda i: (i, 0)),
        compiler_params=pltpu.CompilerParams(
            dimension_semantics=("parallel",)),
    )(z, w1t, sb1, w2t, b2)


def _unshuffle(x):
    B, C, H2, W2 = x.shape
    H, W = H2 // 2, W2 // 2
    x = x.reshape(B, C, H, 2, W, 2).transpose(0, 3, 5, 1, 2, 4)
    return x.reshape(B, 4 * C, H, W)


def _fold_w(x, taps, out_w):
    B, C, H, W = x.shape
    q = jnp.concatenate([x[:, :, :, t:t + out_w] for t in range(taps)], axis=1)
    return q.reshape(B, taps * C, H * out_w)


def kernel(x,
           enc0_w_rows, enc0_sb, enc1_w_rows, enc1_sb,
           enc2_w_rows, enc2_sb, enc3_w_rows, enc3_sb,
           dec0_w_rows, dec0_sb, dec1_w_rows, dec1_sb,
           dec2_w_rows, dec2_sb, dec3_w_rows, dec3_sb,
           mlp_w1t, mlp_sb1, mlp_w2t, mlp_b2):
    B = x.shape[0]
    bb = next(b for b in (4, 2, 1) if B % b == 0)
    h = x.astype(jnp.bfloat16)

    enc = [(enc0_w_rows, enc0_sb, 32, 0), (enc1_w_rows, enc1_sb, 16, 4),
           (enc2_w_rows, enc2_sb, 8, 2), (enc3_w_rows, enc3_sb, 4, 1)]
    for (w_r, sb, k, F) in enc:
        kh = k // 2
        pad = kh - 1
        oh, ow = h.shape[2] // 2, h.shape[3] // 2
        xp = jnp.pad(h, ((0, 0), (0, 0), (pad, pad), (pad, pad)))
        xs = _unshuffle(xp)
        A, co4, KC = w_r.shape
        C4 = KC // kh
        if F == 0:
            ws = ow + kh - 1
            xs_f = xs.reshape(B, xs.shape[1], xs.shape[2] * xs.shape[3])
            xs_f = jnp.pad(xs_f, ((0, 0), (0, 0), (0, kh))).astype(jnp.bfloat16)
            wm = w_r.reshape(A, co4, kh, C4).transpose(2, 1, 0, 3).reshape(
                kh * co4, kh * C4)
            y = _scratch_call(xs_f, wm, sb, taps=kh, oh=oh, ws=ws,
                              relu=True, out_dtype=jnp.bfloat16, bb=bb)
            h = y.reshape(B, co4, oh, ws)[:, :, :, :ow]
        else:
            G = kh // F
            Wp = ow + kh - F
            xf = _fold_w(xs, F, Wp).astype(jnp.bfloat16)
            wm = w_r.reshape(A, co4, G, F, C4).transpose(0, 2, 1, 3, 4)
            wm = wm.reshape(A * G * co4, F * C4)
            out_len = (oh - 1) * Wp + ow
            y = _direct_call(xf, wm, sb, taps_a=A, groups=G, gshift=F,
                             row_stride=Wp, out_len=out_len, relu=True,
                             out_dtype=jnp.bfloat16, bb=bb)
            if out_len != oh * Wp:
                y = jnp.pad(y, ((0, 0), (0, 0), (0, oh * Wp - out_len)))
            h = y.reshape(B, co4, oh, Wp)[:, :, :, :ow]

    z = h.reshape(B, -1)
    zo = _mlp_call(z, mlp_w1t, mlp_sb1, mlp_w2t, mlp_b2)
    h = zo.reshape(B, 64, 8, 8).astype(jnp.bfloat16)

    dec = [(dec0_w_rows, dec0_sb, 4, -1), (dec1_w_rows, dec1_sb, 8, -1),
           (dec2_w_rows, dec2_sb, 16, -1), (dec3_w_rows, dec3_sb, 32, -1)]
    for i, (w_r, sb, k, F) in enumerate(dec):
        khp = k // 2 + 1
        q = (khp - 1) // 2
        H, W = h.shape[2], h.shape[3]
        relu = i < 3
        odt = jnp.bfloat16 if relu else jnp.float32
        xp = jnp.pad(h, ((0, 0), (0, 0), (q, q), (q, q)))
        A, co4, KC = w_r.shape
        C = KC // khp
        if F == 0:
            ws = W + khp - 1
            xs_f = xp.reshape(B, xp.shape[1], xp.shape[2] * xp.shape[3])
            xs_f = jnp.pad(xs_f, ((0, 0), (0, 0), (0, khp))).astype(jnp.bfloat16)
            wm = w_r.reshape(A, co4, khp, C).transpose(2, 1, 0, 3).reshape(
                khp * co4, khp * C)
            y = _scratch_call(xs_f, wm, sb, taps=khp, oh=H, ws=ws,
                              relu=relu, out_dtype=odt, bb=bb)
            y = y.reshape(B, co4, H, ws)[:, :, :, :W]
        else:
            xf = _fold_w(xp, khp, W).astype(jnp.bfloat16)
            wm = w_r.reshape(A * co4, KC)
            y = _direct_call(xf, wm, sb, taps_a=A, groups=1, gshift=0,
                             row_stride=W, out_len=H * W, relu=relu,
                             out_dtype=odt, bb=bb)
            y = y.reshape(B, co4, H, W)
        co = co4 // 4
        y = y.reshape(B, 2, 2, co, H, W).transpose(0, 3, 4, 1, 5, 2)
        h = y.reshape(B, co, 2 * H, 2 * W)
    return h


# speedup vs baseline: 2.2416x; 2.2416x over previous
import functools

import jax
import jax.numpy as jnp
from jax.experimental import pallas as pl
from jax.experimental.pallas import tpu as pltpu


def _direct_body(xf_ref, w_ref, sb_ref, o_ref, *, taps_a, groups, gshift,
                 co4, row_stride, out_len, relu):
    bb = xf_ref.shape[0]
    for b in range(bb):
        pc = jnp.dot(w_ref[...], xf_ref[b],
                     preferred_element_type=jnp.float32)
        acc = None
        for a in range(taps_a):
            for g in range(groups):
                r0 = (a * groups + g) * co4
                off = a * row_stride + g * gshift
                term = pc[r0:r0 + co4, off: off + out_len]
                acc = term if acc is None else acc + term
        y = acc * sb_ref[:, 0:1] + sb_ref[:, 1:2]
        if relu:
            y = jnp.maximum(y, 0.0)
        o_ref[b] = y.astype(o_ref.dtype)


def _direct_call(xf, wm, sb, *, taps_a, groups, gshift, row_stride, out_len,
                 relu, out_dtype, bb):
    B, K, L = xf.shape
    M = wm.shape[0]
    co4 = M // (taps_a * groups)
    body = functools.partial(_direct_body, taps_a=taps_a, groups=groups,
                             gshift=gshift, co4=co4, row_stride=row_stride,
                             out_len=out_len, relu=relu)
    return pl.pallas_call(
        body,
        out_shape=jax.ShapeDtypeStruct((B, co4, out_len), out_dtype),
        grid=(B // bb,),
        in_specs=[
            pl.BlockSpec((bb, K, L), lambda i: (i, 0, 0)),
            pl.BlockSpec((M, K), lambda i: (0, 0)),
            pl.BlockSpec((co4, 2), lambda i: (0, 0)),
        ],
        out_specs=pl.BlockSpec((bb, co4, out_len), lambda i: (i, 0, 0)),
        compiler_params=pltpu.CompilerParams(
            dimension_semantics=("parallel",)),
    )(xf, wm, sb)


def _scratch_body(xs_ref, w_ref, sb_ref, o_ref, xr_ref, *, taps, cdepth,
                  co4, oh, ws, nb, relu):
    ll = oh * ws
    lt = ll + taps - 1
    bb = xs_ref.shape[0]
    for b in range(bb):
        for t in range(taps):
            xr_ref[t * cdepth:(t + 1) * cdepth, :] = (
                xs_ref[b, :, t * ws: t * ws + lt])
        n0 = 0
        while n0 < ll:
            nw = min(nb, ll - n0)
            pc = jnp.dot(w_ref[...], xr_ref[:, n0: n0 + nw + taps - 1],
                         preferred_element_type=jnp.float32)
            acc = pc[0:co4, 0:nw]
            for c in range(1, taps):
                acc = acc + pc[c * co4:(c + 1) * co4, c: c + nw]
            y = acc * sb_ref[:, 0:1] + sb_ref[:, 1:2]
            if relu:
                y = jnp.maximum(y, 0.0)
            o_ref[b, :, n0: n0 + nw] = y.astype(o_ref.dtype)
            n0 += nw


def _scratch_call(xs, wm, sb, *, taps, oh, ws, relu, out_dtype, bb):
    B, C, FL = xs.shape
    M, K = wm.shape
    co4 = M // taps
    ll = oh * ws
    lt = ll + taps - 1
    nb = max(128, (600_000 // (M * 4)) // 128 * 128)
    body = functools.partial(_scratch_body, taps=taps, cdepth=C, co4=co4,
                             oh=oh, ws=ws, nb=nb, relu=relu)
    return pl.pallas_call(
        body,
        out_shape=jax.ShapeDtypeStruct((B, co4, ll), out_dtype),
        grid=(B // bb,),
        in_specs=[
            pl.BlockSpec((bb, C, FL), lambda i: (i, 0, 0)),
            pl.BlockSpec((M, K), lambda i: (0, 0)),
            pl.BlockSpec((co4, 2), lambda i: (0, 0)),
        ],
        out_specs=pl.BlockSpec((bb, co4, ll), lambda i: (i, 0, 0)),
        scratch_shapes=[pltpu.VMEM((K, lt), jnp.bfloat16)],
        compiler_params=pltpu.CompilerParams(
            dimension_semantics=("parallel",)),
    )(xs, wm, sb)


def _mlp_body(z_ref, w1_ref, sb1_ref, w2_ref, b2_ref, o_ref):
    h = jnp.dot(z_ref[...], w1_ref[...], preferred_element_type=jnp.float32)
    h = h * sb1_ref[0:1, :] + sb1_ref[1:2, :]
    h = jnp.maximum(h, 0.0).astype(w2_ref.dtype)
    o_ref[...] = jnp.dot(h, w2_ref[...],
                         preferred_element_type=jnp.float32) + b2_ref[...]


def _mlp_call(z, w1t, sb1, w2t, b2):
    B = z.shape[0]
    gb = B // 2 if B % 2 == 0 else B
    return pl.pallas_call(
        _mlp_body,
        out_shape=jax.ShapeDtypeStruct((B, w2t.shape[1]), jnp.float32),
        grid=(B // gb,),
        in_specs=[
            pl.BlockSpec((gb, w1t.shape[0]), lambda i: (i, 0)),
            pl.BlockSpec(w1t.shape, lambda i: (0, 0)),
            pl.BlockSpec(sb1.shape, lambda i: (0, 0)),
            pl.BlockSpec(w2t.shape, lambda i: (0, 0)),
            pl.BlockSpec(b2.shape, lambda i: (0, 0)),
        ],
        out_specs=pl.BlockSpec((gb, w2t.shape[1]), lambda i: (i, 0)),
        compiler_params=pltpu.CompilerParams(
            dimension_semantics=("parallel",)),
    )(z, w1t, sb1, w2t, b2)


def _unshuffle(x):
    B, C, H2, W2 = x.shape
    H, W = H2 // 2, W2 // 2
    x = x.reshape(B, C, H, 2, W, 2).transpose(0, 3, 5, 1, 2, 4)
    return x.reshape(B, 4 * C, H, W)


def _fold_w(x, taps, out_w):
    B, C, H, W = x.shape
    q = jnp.concatenate([x[:, :, :, t:t + out_w] for t in range(taps)], axis=1)
    return q.reshape(B, taps * C, H * out_w)


def kernel(x,
           enc0_w_rows, enc0_sb, enc1_w_rows, enc1_sb,
           enc2_w_rows, enc2_sb, enc3_w_rows, enc3_sb,
           dec0_w_rows, dec0_sb, dec1_w_rows, dec1_sb,
           dec2_w_rows, dec2_sb, dec3_w_rows, dec3_sb,
           mlp_w1t, mlp_sb1, mlp_w2t, mlp_b2):
    B = x.shape[0]
    bb = next(b for b in (4, 2, 1) if B % b == 0)
    h = x.astype(jnp.bfloat16)

    enc = [(enc0_w_rows, enc0_sb, 32, 16), (enc1_w_rows, enc1_sb, 16, 4),
           (enc2_w_rows, enc2_sb, 8, 2), (enc3_w_rows, enc3_sb, 4, 1)]
    for (w_r, sb, k, F) in enc:
        kh = k // 2
        pad = kh - 1
        oh, ow = h.shape[2] // 2, h.shape[3] // 2
        xp = jnp.pad(h, ((0, 0), (0, 0), (pad, pad), (pad, pad)))
        xs = _unshuffle(xp)
        A, co4, KC = w_r.shape
        C4 = KC // kh
        if F == 0:
            ws = ow + kh - 1
            xs_f = xs.reshape(B, xs.shape[1], xs.shape[2] * xs.shape[3])
            xs_f = jnp.pad(xs_f, ((0, 0), (0, 0), (0, kh))).astype(jnp.bfloat16)
            wm = w_r.reshape(A, co4, kh, C4).transpose(2, 1, 0, 3).reshape(
                kh * co4, kh * C4)
            y = _scratch_call(xs_f, wm, sb, taps=kh, oh=oh, ws=ws,
                              relu=True, out_dtype=jnp.bfloat16, bb=bb)
            h = y.reshape(B, co4, oh, ws)[:, :, :, :ow]
        else:
            G = kh // F
            Wp = ow + kh - F
            xf = _fold_w(xs, F, Wp).astype(jnp.bfloat16)
            wm = w_r.reshape(A, co4, G, F, C4).transpose(0, 2, 1, 3, 4)
            wm = wm.reshape(A * G * co4, F * C4)
            out_len = (oh - 1) * Wp + ow
            y = _direct_call(xf, wm, sb, taps_a=A, groups=G, gshift=F,
                             row_stride=Wp, out_len=out_len, relu=True,
                             out_dtype=jnp.bfloat16, bb=bb)
            if out_len != oh * Wp:
                y = jnp.pad(y, ((0, 0), (0, 0), (0, oh * Wp - out_len)))
            h = y.reshape(B, co4, oh, Wp)[:, :, :, :ow]

    z = h.reshape(B, -1)
    zo = _mlp_call(z, mlp_w1t, mlp_sb1, mlp_w2t, mlp_b2)
    h = zo.reshape(B, 64, 8, 8).astype(jnp.bfloat16)

    dec = [(dec0_w_rows, dec0_sb, 4, -1), (dec1_w_rows, dec1_sb, 8, -1),
           (dec2_w_rows, dec2_sb, 16, -1), (dec3_w_rows, dec3_sb, 32, -1)]
    for i, (w_r, sb, k, F) in enumerate(dec):
        khp = k // 2 + 1
        q = (khp - 1) // 2
        H, W = h.shape[2], h.shape[3]
        relu = i < 3
        odt = jnp.bfloat16 if relu else jnp.float32
        xp = jnp.pad(h, ((0, 0), (0, 0), (q, q), (q, q)))
        A, co4, KC = w_r.shape
        C = KC // khp
        if F == 0:
            ws = W + khp - 1
            xs_f = xp.reshape(B, xp.shape[1], xp.shape[2] * xp.shape[3])
            xs_f = jnp.pad(xs_f, ((0, 0), (0, 0), (0, khp))).astype(jnp.bfloat16)
            wm = w_r.reshape(A, co4, khp, C).transpose(2, 1, 0, 3).reshape(
                khp * co4, khp * C)
            y = _scratch_call(xs_f, wm, sb, taps=khp, oh=H, ws=ws,
                              relu=relu, out_dtype=odt, bb=bb)
            y = y.reshape(B, co4, H, ws)[:, :, :, :W]
        else:
            xf = _fold_w(xp, khp, W).astype(jnp.bfloat16)
            wm = w_r.reshape(A * co4, KC)
            y = _direct_call(xf, wm, sb, taps_a=A, groups=1, gshift=0,
                             row_stride=W, out_len=H * W, relu=relu,
                             out_dtype=odt, bb=bb)
            y = y.reshape(B, co4, H, W)
        co = co4 // 4
        y = y.reshape(B, 2, 2, co, H, W).transpose(0, 3, 4, 1, 5, 2)
        h = y.reshape(B, co, 2 * H, 2 * W)
    return h


# speedup vs baseline: 2.7456x; 1.2248x over previous
import functools

import jax
import jax.numpy as jnp
from jax.experimental import pallas as pl
from jax.experimental.pallas import tpu as pltpu


def _direct_body(xf_ref, w_ref, sb_ref, o_ref, *, taps_a, groups, gshift,
                 co4, row_stride, out_len, relu):
    bb = xf_ref.shape[0]
    for b in range(bb):
        pc = jnp.dot(w_ref[...], xf_ref[b],
                     preferred_element_type=jnp.float32)
        acc = None
        for a in range(taps_a):
            for g in range(groups):
                r0 = (a * groups + g) * co4
                off = a * row_stride + g * gshift
                term = pc[r0:r0 + co4, off: off + out_len]
                acc = term if acc is None else acc + term
        y = acc * sb_ref[:, 0:1] + sb_ref[:, 1:2]
        if relu:
            y = jnp.maximum(y, 0.0)
        o_ref[b] = y.astype(o_ref.dtype)


def _direct_call(xf, wm, sb, *, taps_a, groups, gshift, row_stride, out_len,
                 relu, out_dtype, bb):
    B, K, L = xf.shape
    M = wm.shape[0]
    co4 = M // (taps_a * groups)
    body = functools.partial(_direct_body, taps_a=taps_a, groups=groups,
                             gshift=gshift, co4=co4, row_stride=row_stride,
                             out_len=out_len, relu=relu)
    return pl.pallas_call(
        body,
        out_shape=jax.ShapeDtypeStruct((B, co4, out_len), out_dtype),
        grid=(B // bb,),
        in_specs=[
            pl.BlockSpec((bb, K, L), lambda i: (i, 0, 0)),
            pl.BlockSpec((M, K), lambda i: (0, 0)),
            pl.BlockSpec((co4, 2), lambda i: (0, 0)),
        ],
        out_specs=pl.BlockSpec((bb, co4, out_len), lambda i: (i, 0, 0)),
        compiler_params=pltpu.CompilerParams(
            dimension_semantics=("parallel",)),
    )(xf, wm, sb)


def _scratch_body(xs_ref, w_ref, sb_ref, o_ref, xr_ref, *, taps, cdepth,
                  co4, oh, ws, nb, relu):
    ll = oh * ws
    lt = ll + taps - 1
    bb = xs_ref.shape[0]
    for b in range(bb):
        for t in range(taps):
            xr_ref[t * cdepth:(t + 1) * cdepth, :] = (
                xs_ref[b, :, t * ws: t * ws + lt])
        n0 = 0
        while n0 < ll:
            nw = min(nb, ll - n0)
            pc = jnp.dot(w_ref[...], xr_ref[:, n0: n0 + nw + taps - 1],
                         preferred_element_type=jnp.float32)
            acc = pc[0:co4, 0:nw]
            for c in range(1, taps):
                acc = acc + pc[c * co4:(c + 1) * co4, c: c + nw]
            y = acc * sb_ref[:, 0:1] + sb_ref[:, 1:2]
            if relu:
                y = jnp.maximum(y, 0.0)
            o_ref[b, :, n0: n0 + nw] = y.astype(o_ref.dtype)
            n0 += nw


def _scratch_call(xs, wm, sb, *, taps, oh, ws, relu, out_dtype, bb):
    B, C, FL = xs.shape
    M, K = wm.shape
    co4 = M // taps
    ll = oh * ws
    lt = ll + taps - 1
    nb = max(128, (600_000 // (M * 4)) // 128 * 128)
    body = functools.partial(_scratch_body, taps=taps, cdepth=C, co4=co4,
                             oh=oh, ws=ws, nb=nb, relu=relu)
    return pl.pallas_call(
        body,
        out_shape=jax.ShapeDtypeStruct((B, co4, ll), out_dtype),
        grid=(B // bb,),
        in_specs=[
            pl.BlockSpec((bb, C, FL), lambda i: (i, 0, 0)),
            pl.BlockSpec((M, K), lambda i: (0, 0)),
            pl.BlockSpec((co4, 2), lambda i: (0, 0)),
        ],
        out_specs=pl.BlockSpec((bb, co4, ll), lambda i: (i, 0, 0)),
        scratch_shapes=[pltpu.VMEM((K, lt), jnp.bfloat16)],
        compiler_params=pltpu.CompilerParams(
            dimension_semantics=("parallel",)),
    )(xs, wm, sb)


def _repack_body(x4_ref, o_ref, *, rows, w, slack):
    bb = x4_ref.shape[0]
    for b in range(bb):
        for u in range(rows):
            o_ref[b, :, u * w:(u + 1) * w] = x4_ref[b, :, u, :]
        if slack:
            o_ref[b, :, rows * w:] = jnp.zeros(
                (x4_ref.shape[1], slack), o_ref.dtype)


def _repack_call(x4, slack, *, bb):
    B, C, H, W = x4.shape
    body = functools.partial(_repack_body, rows=H, w=W, slack=slack)
    return pl.pallas_call(
        body,
        out_shape=jax.ShapeDtypeStruct((B, C, H * W + slack), x4.dtype),
        grid=(B // bb,),
        in_specs=[pl.BlockSpec((bb, C, H, W), lambda i: (i, 0, 0, 0))],
        out_specs=pl.BlockSpec((bb, C, H * W + slack),
                               lambda i: (i, 0, 0)),
        compiler_params=pltpu.CompilerParams(
            dimension_semantics=("parallel",)),
    )(x4)


def _mlp_body(z_ref, w1_ref, sb1_ref, w2_ref, b2_ref, o_ref):
    h = jnp.dot(z_ref[...], w1_ref[...], preferred_element_type=jnp.float32)
    h = h * sb1_ref[0:1, :] + sb1_ref[1:2, :]
    h = jnp.maximum(h, 0.0).astype(w2_ref.dtype)
    o_ref[...] = jnp.dot(h, w2_ref[...],
                         preferred_element_type=jnp.float32) + b2_ref[...]


def _mlp_call(z, w1t, sb1, w2t, b2):
    B = z.shape[0]
    gb = B // 2 if B % 2 == 0 else B
    return pl.pallas_call(
        _mlp_body,
        out_shape=jax.ShapeDtypeStruct((B, w2t.shape[1]), jnp.float32),
        grid=(B // gb,),
        in_specs=[
            pl.BlockSpec((gb, w1t.shape[0]), lambda i: (i, 0)),
            pl.BlockSpec(w1t.shape, lambda i: (0, 0)),
            pl.BlockSpec(sb1.shape, lambda i: (0, 0)),
            pl.BlockSpec(w2t.shape, lambda i: (0, 0)),
            pl.BlockSpec(b2.shape, lambda i: (0, 0)),
        ],
        out_specs=pl.BlockSpec((gb, w2t.shape[1]), lambda i: (i, 0)),
        compiler_params=pltpu.CompilerParams(
            dimension_semantics=("parallel",)),
    )(z, w1t, sb1, w2t, b2)


def _unshuffle(x):
    B, C, H2, W2 = x.shape
    H, W = H2 // 2, W2 // 2
    x = x.reshape(B, C, H, 2, W, 2).transpose(0, 3, 5, 1, 2, 4)
    return x.reshape(B, 4 * C, H, W)


def _fold_w(x, taps, out_w):
    B, C, H, W = x.shape
    q = jnp.concatenate([x[:, :, :, t:t + out_w] for t in range(taps)], axis=1)
    return q.reshape(B, taps * C, H * out_w)


def kernel(x,
           enc0_w_rows, enc0_sb, enc1_w_rows, enc1_sb,
           enc2_w_rows, enc2_sb, enc3_w_rows, enc3_sb,
           dec0_w_rows, dec0_sb, dec1_w_rows, dec1_sb,
           dec2_w_rows, dec2_sb, dec3_w_rows, dec3_sb,
           mlp_w1t, mlp_sb1, mlp_w2t, mlp_b2):
    B = x.shape[0]
    bb = next(b for b in (4, 2, 1) if B % b == 0)
    h = x.astype(jnp.bfloat16)

    enc = [(enc0_w_rows, enc0_sb, 32, 16), (enc1_w_rows, enc1_sb, 16, 4),
           (enc2_w_rows, enc2_sb, 8, 2), (enc3_w_rows, enc3_sb, 4, 1)]
    for (w_r, sb, k, F) in enc:
        kh = k // 2
        pad = kh - 1
        oh, ow = h.shape[2] // 2, h.shape[3] // 2
        xp = jnp.pad(h, ((0, 0), (0, 0), (pad, pad), (pad, pad)))
        xs = _unshuffle(xp)
        A, co4, KC = w_r.shape
        C4 = KC // kh
        if F == 0:
            ws = ow + kh - 1
            xs_f = xs.reshape(B, xs.shape[1], xs.shape[2] * xs.shape[3])
            xs_f = jnp.pad(xs_f, ((0, 0), (0, 0), (0, kh))).astype(jnp.bfloat16)
            wm = w_r.reshape(A, co4, kh, C4).transpose(2, 1, 0, 3).reshape(
                kh * co4, kh * C4)
            y = _scratch_call(xs_f, wm, sb, taps=kh, oh=oh, ws=ws,
                              relu=True, out_dtype=jnp.bfloat16, bb=bb)
            h = y.reshape(B, co4, oh, ws)[:, :, :, :ow]
        else:
            G = kh // F
            Wp = ow + kh - F
            xf = _fold_w(xs, F, Wp).astype(jnp.bfloat16)
            wm = w_r.reshape(A, co4, G, F, C4).transpose(0, 2, 1, 3, 4)
            wm = wm.reshape(A * G * co4, F * C4)
            out_len = (oh - 1) * Wp + ow
            y = _direct_call(xf, wm, sb, taps_a=A, groups=G, gshift=F,
                             row_stride=Wp, out_len=out_len, relu=True,
                             out_dtype=jnp.bfloat16, bb=bb)
            if out_len != oh * Wp:
                y = jnp.pad(y, ((0, 0), (0, 0), (0, oh * Wp - out_len)))
            h = y.reshape(B, co4, oh, Wp)[:, :, :, :ow]

    z = h.reshape(B, -1)
    zo = _mlp_call(z, mlp_w1t, mlp_sb1, mlp_w2t, mlp_b2)
    h = zo.reshape(B, 64, 8, 8).astype(jnp.bfloat16)

    dec = [(dec0_w_rows, dec0_sb, 4, -1), (dec1_w_rows, dec1_sb, 8, -1),
           (dec2_w_rows, dec2_sb, 16, -1), (dec3_w_rows, dec3_sb, 32, 0)]
    for i, (w_r, sb, k, F) in enumerate(dec):
        khp = k // 2 + 1
        q = (khp - 1) // 2
        H, W = h.shape[2], h.shape[3]
        relu = i < 3
        odt = jnp.bfloat16 if relu else jnp.float32
        xp = jnp.pad(h, ((0, 0), (0, 0), (q, q), (q, q)))
        A, co4, KC = w_r.shape
        C = KC // khp
        if F == 0:
            ws = W + khp - 1
            xs_f = _repack_call(xp.astype(jnp.bfloat16), khp, bb=bb)
            wm = w_r.reshape(A, co4, khp, C).transpose(2, 1, 0, 3).reshape(
                khp * co4, khp * C)
            y = _scratch_call(xs_f, wm, sb, taps=khp, oh=H, ws=ws,
                              relu=relu, out_dtype=odt, bb=bb)
            y = y.reshape(B, co4, H, ws)[:, :, :, :W]
        else:
            xf = _fold_w(xp, khp, W).astype(jnp.bfloat16)
            wm = w_r.reshape(A * co4, KC)
            y = _direct_call(xf, wm, sb, taps_a=A, groups=1, gshift=0,
                             row_stride=W, out_len=H * W, relu=relu,
                             out_dtype=odt, bb=bb)
            y = y.reshape(B, co4, H, W)
        co = co4 // 4
        y = y.reshape(B, 2, 2, co, H, W).transpose(0, 3, 4, 1, 5, 2)
        h = y.reshape(B, co, 2 * H, 2 * W)
    return h
